# baseline (speedup 1.0000x reference)
import jax
import jax.numpy as jnp
from jax.experimental import pallas as pl
from jax.experimental.pallas import tpu as pltpu

_VOCAB = 1_000_000
_EMB = 64
_NCHUNKS = 8
_CHUNK = _VOCAB // _NCHUNKS


def _dma_body(in_hbm, out_hbm, sems):
    copies = [
        pltpu.make_async_copy(
            in_hbm.at[pl.ds(i * _CHUNK, _CHUNK), :],
            out_hbm.at[pl.ds(i * _CHUNK, _CHUNK), :],
            sems.at[i],
        )
        for i in range(_NCHUNKS)
    ]
    for c in copies:
        c.start()
    for c in copies:
        c.wait()


def kernel(lang, W_emb):
    del lang
    out = pl.pallas_call(
        _dma_body,
        in_specs=[pl.BlockSpec(memory_space=pltpu.MemorySpace.HBM)],
        out_specs=pl.BlockSpec(memory_space=pltpu.MemorySpace.HBM),
        out_shape=jax.ShapeDtypeStruct((_VOCAB, _EMB), jnp.float32),
        scratch_shapes=[pltpu.SemaphoreType.DMA((_NCHUNKS,))],
    )(W_emb)
    return out


# speedup vs baseline: 16.0493x; 16.0493x over previous
import jax
import jax.numpy as jnp
from jax.experimental import pallas as pl
from jax.experimental.pallas import tpu as pltpu

_VOCAB = 1_000_000
_EMB = 64
_NCHUNKS = 125
_CHUNK = _VOCAB // _NCHUNKS
_NBUF = 4


def _dma_body(in_hbm, out_hbm, bufs, in_sems, out_sems):
    def in_copy(i, slot):
        return pltpu.make_async_copy(
            in_hbm.at[pl.ds(i * _CHUNK, _CHUNK), :],
            bufs.at[slot],
            in_sems.at[slot],
        )

    def out_copy(i, slot):
        return pltpu.make_async_copy(
            bufs.at[slot],
            out_hbm.at[pl.ds(i * _CHUNK, _CHUNK), :],
            out_sems.at[slot],
        )

    for i in range(_NBUF):
        in_copy(i, i).start()

    def step(i, _):
        slot = jax.lax.rem(i, _NBUF)
        in_copy(i, slot).wait()
        out_copy(i, slot).start()

        @pl.when(i + _NBUF < _NCHUNKS)
        def _():
            out_copy(i, slot).wait()
            in_copy(i + _NBUF, slot).start()

        return ()

    jax.lax.fori_loop(0, _NCHUNKS, step, ())
    for i in range(_NCHUNKS - _NBUF, _NCHUNKS):
        out_copy(i, jax.lax.rem(i, _NBUF)).wait()


def kernel(lang, W_emb):
    del lang
    out = pl.pallas_call(
        _dma_body,
        in_specs=[pl.BlockSpec(memory_space=pltpu.MemorySpace.HBM)],
        out_specs=pl.BlockSpec(memory_space=pltpu.MemorySpace.HBM),
        out_shape=jax.ShapeDtypeStruct((_VOCAB, _EMB), jnp.float32),
        scratch_shapes=[
            pltpu.VMEM((_NBUF, _CHUNK, _EMB), jnp.float32),
            pltpu.SemaphoreType.DMA((_NBUF,)),
            pltpu.SemaphoreType.DMA((_NBUF,)),
        ],
    )(W_emb)
    return out
